# baseline (speedup 1.0000x reference)
import functools

import jax
import jax.numpy as jnp
from jax import lax
from jax.experimental import pallas as pl
from jax.experimental.pallas import tpu as pltpu
from jax.experimental.pallas import tpu_sc as plsc

_NUM_CORES = 2
_NUM_SUBCORES = 16
_NW = _NUM_CORES * _NUM_SUBCORES

_CHUNK = 16
_NBUF = 4


@functools.lru_cache(maxsize=None)
def _make_gather(total: int, hidden: int):
    assert total % (_NW * _NBUF * _CHUNK) == 0
    b_per_w = total // _NW
    n_chunks = b_per_w // _CHUNK
    n_groups = n_chunks // _NBUF

    mesh = plsc.VectorSubcoreMesh(core_axis_name="c", subcore_axis_name="s")

    scratch = [pltpu.VMEM((n_chunks, _CHUNK), jnp.int32)]
    scratch += [pltpu.VMEM((_CHUNK, hidden), jnp.float32)
                for _ in range(_NBUF)]
    scratch += [pltpu.VMEM_SHARED((_NUM_SUBCORES * 3 * _CHUNK, hidden),
                                  jnp.float32)]
    scratch += [pltpu.SemaphoreType.DMA for _ in range(_NBUF + 6)]

    @functools.partial(
        pl.kernel,
        mesh=mesh,
        out_type=jax.ShapeDtypeStruct((total, hidden), jnp.float32),
        scratch_types=scratch,
    )
    def gather_kernel(idx_hbm, table_hbm, out_hbm, idx_v, *rest):
        bufs = rest[:_NBUF]
        shared = rest[_NBUF]
        sg = rest[_NBUF + 1:2 * _NBUF + 1]
        sx = rest[2 * _NBUF + 1:2 * _NBUF + 4]
        so = rest[2 * _NBUF + 4:2 * _NBUF + 7]

        cid = lax.axis_index("c")
        sid = lax.axis_index("s")
        wid = sid * _NUM_CORES + cid
        base = wid * b_per_w

        pltpu.sync_copy(idx_hbm.at[wid], idx_v)

        def slot(s):
            return shared.at[pl.ds((sid * 3 + s) * _CHUNK, _CHUNK)]

        def gather_start(c, buf, sem):
            pltpu.async_copy(table_hbm.at[idx_v.at[c]], buf, sem)

        def gather_wait(c, buf, sem):
            pltpu.make_async_copy(table_hbm.at[idx_v.at[c]], buf, sem).wait()

        def xbar_start(s, buf):
            pltpu.async_copy(buf, slot(s), sx[s])

        def xbar_wait(s, buf):
            pltpu.make_async_copy(buf, slot(s), sx[s]).wait()

        def out_start(c, s):
            pltpu.async_copy(slot(s),
                             out_hbm.at[pl.ds(base + c * _CHUNK, _CHUNK)],
                             so[s])

        def out_wait(s):
            pltpu.make_async_copy(slot(s), out_hbm.at[pl.ds(base, _CHUNK)],
                                  so[s]).wait()

        for b in range(_NBUF):
            gather_start(b, bufs[b], sg[b])

        def group_body(g, carry):
            c0 = g * _NBUF
            for b in range(_NBUF):
                s = b % 3
                gather_wait(c0 + b, bufs[b], sg[b])

                if b < 3:
                    @pl.when(g > 0)
                    def _(s=s):
                        out_wait(s)
                else:
                    out_wait(s)

                xbar_start(s, bufs[b])
                xbar_wait(s, bufs[b])
                out_start(c0 + b, s)

                @pl.when(g + 1 < n_groups)
                def _(b=b, c0=c0):
                    gather_start(c0 + _NBUF + b, bufs[b], sg[b])

            return carry

        lax.fori_loop(0, n_groups, group_body, 0)

        for s in range(3):
            out_wait(s)

    return gather_kernel


def kernel(position_ids, embedding_table):
    batch, seq = position_ids.shape
    _, hidden = embedding_table.shape
    total = batch * seq

    b_per_w = total // _NW
    n_chunks = b_per_w // _CHUNK
    ids = position_ids.astype(jnp.int32).reshape(_NW, n_chunks, _CHUNK)
    table = embedding_table.astype(jnp.float32)

    out = _make_gather(total, hidden)(ids, table)
    return out.reshape(batch, seq, hidden)


# speedup vs baseline: 1.0079x; 1.0079x over previous
import functools

import jax
import jax.numpy as jnp
from jax import lax
from jax.experimental import pallas as pl
from jax.experimental.pallas import tpu as pltpu
from jax.experimental.pallas import tpu_sc as plsc

_NUM_CORES = 2
_NUM_SUBCORES = 16
_NW = _NUM_CORES * _NUM_SUBCORES

_CHUNK = 16
_NBUF = 4


@functools.lru_cache(maxsize=None)
def _make_gather(total: int, hidden: int):
    assert total % (_NW * _NBUF * _CHUNK) == 0
    b_per_w = total // _NW
    n_chunks = b_per_w // _CHUNK
    n_groups = n_chunks // _NBUF

    mesh = plsc.VectorSubcoreMesh(core_axis_name="c", subcore_axis_name="s")

    scratch = [pltpu.VMEM((n_chunks, _CHUNK), jnp.int32)]
    scratch += [pltpu.VMEM((_CHUNK, hidden), jnp.float32)
                for _ in range(_NBUF)]
    scratch += [pltpu.VMEM_SHARED((_NUM_SUBCORES * 2 * _CHUNK, hidden),
                                  jnp.float32)]
    scratch += [pltpu.SemaphoreType.DMA for _ in range(_NBUF + 4)]

    @functools.partial(
        pl.kernel,
        mesh=mesh,
        out_type=jax.ShapeDtypeStruct((total, hidden), jnp.float32),
        scratch_types=scratch,
    )
    def gather_kernel(idx_hbm, table_hbm, out_hbm, idx_v, *rest):
        bufs = rest[:_NBUF]
        shared = rest[_NBUF]
        sg = rest[_NBUF + 1:2 * _NBUF + 1]
        sx = rest[2 * _NBUF + 1:2 * _NBUF + 3]
        so = rest[2 * _NBUF + 3:2 * _NBUF + 5]

        cid = lax.axis_index("c")
        sid = lax.axis_index("s")
        wid = sid * _NUM_CORES + cid
        base = wid * b_per_w

        pltpu.sync_copy(idx_hbm.at[wid], idx_v)

        def slot(s):
            return shared.at[pl.ds((sid * 2 + s) * _CHUNK, _CHUNK)]

        def gather_start(c, buf, sem):
            pltpu.async_copy(table_hbm.at[idx_v.at[c]], buf, sem)

        def gather_wait(c, buf, sem):
            pltpu.make_async_copy(table_hbm.at[idx_v.at[c]], buf, sem).wait()

        def xbar_start(s, buf):
            pltpu.async_copy(buf, slot(s), sx[s])

        def xbar_wait(s, buf):
            pltpu.make_async_copy(buf, slot(s), sx[s]).wait()

        def out_start(c, s):
            pltpu.async_copy(slot(s),
                             out_hbm.at[pl.ds(base + c * _CHUNK, _CHUNK)],
                             so[s])

        def out_wait(s):
            pltpu.make_async_copy(slot(s), out_hbm.at[pl.ds(base, _CHUNK)],
                                  so[s]).wait()

        for b in range(_NBUF):
            gather_start(b, bufs[b], sg[b])

        def group_body(g, carry):
            c0 = g * _NBUF
            for b in range(_NBUF):
                s = b % 2
                gather_wait(c0 + b, bufs[b], sg[b])

                if b < 2:
                    @pl.when(g > 0)
                    def _(s=s):
                        out_wait(s)
                else:
                    out_wait(s)

                xbar_start(s, bufs[b])
                xbar_wait(s, bufs[b])
                out_start(c0 + b, s)

                @pl.when(g + 1 < n_groups)
                def _(b=b, c0=c0):
                    gather_start(c0 + _NBUF + b, bufs[b], sg[b])

            return carry

        lax.fori_loop(0, n_groups, group_body, 0)

        for s in range(2):
            out_wait(s)

    return gather_kernel


def kernel(position_ids, embedding_table):
    batch, seq = position_ids.shape
    _, hidden = embedding_table.shape
    total = batch * seq

    b_per_w = total // _NW
    n_chunks = b_per_w // _CHUNK
    ids = position_ids.astype(jnp.int32).reshape(_NW, n_chunks, _CHUNK)
    table = embedding_table.astype(jnp.float32)

    out = _make_gather(total, hidden)(ids, table)
    return out.reshape(batch, seq, hidden)
